# baseline (speedup 1.0000x reference)
import functools

import jax
import jax.numpy as jnp
from jax import lax
from jax.experimental import pallas as pl
from jax.experimental.pallas import tpu as pltpu
from jax.experimental.pallas import tpu_sc as plsc

BQ = 1024
BK = 1000
CHUNK_BLOCKS = 5


def _round_bf16(v):
    u = lax.bitcast_convert_type(v, jnp.uint32)
    r = (u + jnp.uint32(0x7FFF) + ((u >> jnp.uint32(16)) & jnp.uint32(1))) \
        & jnp.uint32(0xFFFF0000)
    return lax.bitcast_convert_type(r, jnp.float32)


def _argmin_body(xsq_ref, xm2_ref, p_ref, psq_ref, idx_ref,
                 cmin_ref, cidx_ref, rmin_ref, ridx_ref):
    ki = pl.program_id(1)
    nk = pl.num_programs(1)

    @pl.when(ki == 0)
    def _init():
        rmin_ref[...] = jnp.full((BQ, 1), jnp.inf, jnp.float32)
        ridx_ref[...] = jnp.zeros((BQ, 1), jnp.int32)

    @pl.when(ki % CHUNK_BLOCKS == 0)
    def _init_chunk():
        cmin_ref[...] = jnp.full((BQ, 1), jnp.inf, jnp.float32)
        cidx_ref[...] = jnp.zeros((BQ, 1), jnp.int32)

    xp = lax.dot_general(
        xm2_ref[...], p_ref[...],
        dimension_numbers=(((1,), (1,)), ((), ())),
        preferred_element_type=jnp.float32,
    )
    dist = (xsq_ref[...] + xp) + psq_ref[...].reshape(1, BK)

    m = jnp.min(dist, axis=1, keepdims=True)
    col = lax.broadcasted_iota(jnp.int32, (BQ, BK), 1)
    idxb = jnp.min(jnp.where(dist == m, col, jnp.int32(2**30)),
                   axis=1, keepdims=True) + ki * BK
    better = m < cmin_ref[...]
    cidx_ref[...] = jnp.where(better, idxb, cidx_ref[...])
    cmin_ref[...] = jnp.where(better, m, cmin_ref[...])

    @pl.when(ki % CHUNK_BLOCKS == CHUNK_BLOCKS - 1)
    def _merge_chunk():
        cm = cmin_ref[...]
        upd = cm < rmin_ref[...]
        ridx_ref[...] = jnp.where(upd, cidx_ref[...], ridx_ref[...])
        rmin_ref[...] = jnp.where(upd, _round_bf16(cm), rmin_ref[...])

    @pl.when(ki == nk - 1)
    def _emit():
        idx_ref[...] = ridx_ref[...]


def _nn_indices(x_sq, xm2, p_pad, psq_pad):
    q = xm2.shape[0]
    k_pad, d = p_pad.shape
    grid = (q // BQ, k_pad // BK)
    return pl.pallas_call(
        _argmin_body,
        grid=grid,
        in_specs=[
            pl.BlockSpec((BQ, 1), lambda qi, ki: (qi, 0)),
            pl.BlockSpec((BQ, d), lambda qi, ki: (qi, 0)),
            pl.BlockSpec((BK, d), lambda qi, ki: (ki, 0)),
            pl.BlockSpec((1, 1, BK), lambda qi, ki: (ki, 0, 0)),
        ],
        out_specs=pl.BlockSpec((BQ, 1), lambda qi, ki: (qi, 0)),
        out_shape=jax.ShapeDtypeStruct((q, 1), jnp.int32),
        scratch_shapes=[
            pltpu.VMEM((BQ, 1), jnp.float32),
            pltpu.VMEM((BQ, 1), jnp.int32),
            pltpu.VMEM((BQ, 1), jnp.float32),
            pltpu.VMEM((BQ, 1), jnp.int32),
        ],
        compiler_params=pltpu.CompilerParams(
            dimension_semantics=("parallel", "arbitrary"),
        ),
    )(x_sq, xm2, p_pad, psq_pad)


def _sc_gather(prototypes, indices):
    q = indices.shape[0]
    d = prototypes.shape[1]
    try:
        info = plsc.get_sparse_core_info()
        nc, ns = info.num_cores, info.num_subcores
    except Exception:
        nc, ns = 2, 16
    nw = nc * ns
    bpw = q // nw
    mesh = plsc.VectorSubcoreMesh(core_axis_name="c", subcore_axis_name="s")

    @functools.partial(
        pl.kernel,
        mesh=mesh,
        out_type=jax.ShapeDtypeStruct((q, d), jnp.float32),
        scratch_types=[
            pltpu.VMEM((bpw,), jnp.int32),
            pltpu.VMEM((bpw, d), jnp.float32),
            pltpu.SemaphoreType.DMA,
        ],
    )
    def gather_kernel(table_hbm, idx_hbm, out_hbm, idx_v, rows_v, sem):
        wid = lax.axis_index("s") * nc + lax.axis_index("c")
        base = wid * bpw
        pltpu.sync_copy(idx_hbm.at[pl.ds(base, bpw)], idx_v)
        pltpu.async_copy(table_hbm.at[idx_v], rows_v, sem).wait()
        pltpu.sync_copy(rows_v, out_hbm.at[pl.ds(base, bpw)])

    return gather_kernel(prototypes, indices)


def kernel(x, prototypes):
    q, d = x.shape
    k = prototypes.shape[0]
    k_pad = ((k + BK - 1) // BK) * BK

    x_sq = jnp.sum(x * x, axis=1, keepdims=True)
    p_sq = jnp.sum(prototypes * prototypes, axis=1)
    xm2 = (x * jnp.float32(-2.0)).astype(jnp.bfloat16)
    p_pad = jnp.pad(prototypes.astype(jnp.bfloat16), ((0, k_pad - k), (0, 0)))
    psq_pad = jnp.pad(p_sq, (0, k_pad - k),
                      constant_values=jnp.inf).reshape(k_pad // BK, 1, BK)

    idx = _nn_indices(x_sq, xm2, p_pad, psq_pad).reshape(q)
    return _sc_gather(prototypes, idx)


# speedup vs baseline: 1.2910x; 1.2910x over previous
import functools

import jax
import jax.numpy as jnp
from jax import lax
from jax.experimental import pallas as pl
from jax.experimental.pallas import tpu as pltpu
from jax.experimental.pallas import tpu_sc as plsc

BQ = 1024
BK = 5000
CHUNK_BLOCKS = 1


def _round_bf16(v):
    u = lax.bitcast_convert_type(v, jnp.uint32)
    r = (u + jnp.uint32(0x7FFF) + ((u >> jnp.uint32(16)) & jnp.uint32(1))) \
        & jnp.uint32(0xFFFF0000)
    return lax.bitcast_convert_type(r, jnp.float32)


def _argmin_body(xsq_ref, xm2_ref, p_ref, psq_ref, idx_ref,
                 cmin_ref, cidx_ref, rmin_ref, ridx_ref):
    ki = pl.program_id(1)
    nk = pl.num_programs(1)

    @pl.when(ki == 0)
    def _init():
        rmin_ref[...] = jnp.full((BQ, 1), jnp.inf, jnp.float32)
        ridx_ref[...] = jnp.zeros((BQ, 1), jnp.int32)

    @pl.when(ki % CHUNK_BLOCKS == 0)
    def _init_chunk():
        cmin_ref[...] = jnp.full((BQ, 1), jnp.inf, jnp.float32)
        cidx_ref[...] = jnp.zeros((BQ, 1), jnp.int32)

    xp = lax.dot_general(
        xm2_ref[...], p_ref[...],
        dimension_numbers=(((1,), (1,)), ((), ())),
        preferred_element_type=jnp.float32,
    )
    dist = (xsq_ref[...] + xp) + psq_ref[...].reshape(1, BK)

    m = jnp.min(dist, axis=1, keepdims=True)
    col = lax.broadcasted_iota(jnp.int32, (BQ, BK), 1)
    idxb = jnp.min(jnp.where(dist == m, col, jnp.int32(2**30)),
                   axis=1, keepdims=True) + ki * BK
    better = m < cmin_ref[...]
    cidx_ref[...] = jnp.where(better, idxb, cidx_ref[...])
    cmin_ref[...] = jnp.where(better, m, cmin_ref[...])

    @pl.when(ki % CHUNK_BLOCKS == CHUNK_BLOCKS - 1)
    def _merge_chunk():
        cm = cmin_ref[...]
        upd = cm < rmin_ref[...]
        ridx_ref[...] = jnp.where(upd, cidx_ref[...], ridx_ref[...])
        rmin_ref[...] = jnp.where(upd, _round_bf16(cm), rmin_ref[...])

    @pl.when(ki == nk - 1)
    def _emit():
        idx_ref[...] = ridx_ref[...]


def _nn_indices(x_sq, xm2, p_pad, psq_pad):
    q = xm2.shape[0]
    k_pad, d = p_pad.shape
    grid = (q // BQ, k_pad // BK)
    return pl.pallas_call(
        _argmin_body,
        grid=grid,
        in_specs=[
            pl.BlockSpec((BQ, 1), lambda qi, ki: (qi, 0)),
            pl.BlockSpec((BQ, d), lambda qi, ki: (qi, 0)),
            pl.BlockSpec((BK, d), lambda qi, ki: (ki, 0)),
            pl.BlockSpec((1, 1, BK), lambda qi, ki: (ki, 0, 0)),
        ],
        out_specs=pl.BlockSpec((BQ, 1), lambda qi, ki: (qi, 0)),
        out_shape=jax.ShapeDtypeStruct((q, 1), jnp.int32),
        scratch_shapes=[
            pltpu.VMEM((BQ, 1), jnp.float32),
            pltpu.VMEM((BQ, 1), jnp.int32),
            pltpu.VMEM((BQ, 1), jnp.float32),
            pltpu.VMEM((BQ, 1), jnp.int32),
        ],
        compiler_params=pltpu.CompilerParams(
            dimension_semantics=("parallel", "arbitrary"),
        ),
    )(x_sq, xm2, p_pad, psq_pad)


def _sc_gather(prototypes, indices):
    q = indices.shape[0]
    d = prototypes.shape[1]
    try:
        info = plsc.get_sparse_core_info()
        nc, ns = info.num_cores, info.num_subcores
    except Exception:
        nc, ns = 2, 16
    nw = nc * ns
    bpw = q // nw
    mesh = plsc.VectorSubcoreMesh(core_axis_name="c", subcore_axis_name="s")

    @functools.partial(
        pl.kernel,
        mesh=mesh,
        out_type=jax.ShapeDtypeStruct((q, d), jnp.float32),
        scratch_types=[
            pltpu.VMEM((bpw,), jnp.int32),
            pltpu.VMEM((bpw, d), jnp.float32),
            pltpu.SemaphoreType.DMA,
        ],
    )
    def gather_kernel(table_hbm, idx_hbm, out_hbm, idx_v, rows_v, sem):
        wid = lax.axis_index("s") * nc + lax.axis_index("c")
        base = wid * bpw
        pltpu.sync_copy(idx_hbm.at[pl.ds(base, bpw)], idx_v)
        pltpu.async_copy(table_hbm.at[idx_v], rows_v, sem).wait()
        pltpu.sync_copy(rows_v, out_hbm.at[pl.ds(base, bpw)])

    return gather_kernel(prototypes, indices)


def kernel(x, prototypes):
    q, d = x.shape
    k = prototypes.shape[0]
    k_pad = ((k + BK - 1) // BK) * BK

    x_sq = jnp.sum(x * x, axis=1, keepdims=True)
    p_sq = jnp.sum(prototypes * prototypes, axis=1)
    xm2 = (x * jnp.float32(-2.0)).astype(jnp.bfloat16)
    p_pad = jnp.pad(prototypes.astype(jnp.bfloat16), ((0, k_pad - k), (0, 0)))
    psq_pad = jnp.pad(p_sq, (0, k_pad - k),
                      constant_values=jnp.inf).reshape(k_pad // BK, 1, BK)

    idx = _nn_indices(x_sq, xm2, p_pad, psq_pad).reshape(q)
    return _sc_gather(prototypes, idx)
